# baseline (speedup 1.0000x reference)
import functools

import jax
import jax.numpy as jnp
from jax.experimental import pallas as pl


_TS = 512
_D = 2048
_E = 64

_KS0 = 0
_KS1 = 42
_KS2 = _KS0 ^ _KS1 ^ 0x1BD11BDA
_ROTS = ((13, 15, 26, 6), (17, 29, 16, 24))
_ORDER = ((1, 2), (2, 0), (0, 1), (1, 2), (2, 0))
_KS = (_KS0, _KS1, _KS2)


def _rotl(x, r):
    return (x << jnp.uint32(r)) | (x >> jnp.uint32(32 - r))


def _router_kernel(hs_ref, w_ref, b_ref, logits_ref, probs_ref, idx_ref):
    t = pl.program_id(0)
    base = (t * (_TS * _D)).astype(jnp.uint32)
    row = jax.lax.broadcasted_iota(jnp.uint32, (_TS, _D), 0)
    col = jax.lax.broadcasted_iota(jnp.uint32, (_TS, _D), 1)
    cnt = base + row * jnp.uint32(_D) + col

    x0 = jnp.zeros((_TS, _D), jnp.uint32) + jnp.uint32(_KS0)
    x1 = cnt + jnp.uint32(_KS1)
    for i in range(5):
        for r in _ROTS[i % 2]:
            x0 = x0 + x1
            x1 = _rotl(x1, r)
            x1 = x1 ^ x0
        a, b = _ORDER[i]
        x0 = x0 + jnp.uint32(_KS[a])
        x1 = x1 + jnp.uint32(_KS[b]) + jnp.uint32(i + 1)
    bits = x0 ^ x1

    u = jax.lax.bitcast_convert_type(
        (bits >> jnp.uint32(9)) | jnp.uint32(0x3F800000), jnp.float32) - 1.0
    noise = u * jnp.float32(-0.02) + jnp.float32(1.01)

    new_attr = hs_ref[...] * noise
    logits = jax.lax.dot_general(
        new_attr, w_ref[...], (((1,), (0,)), ((), ())),
        preferred_element_type=jnp.float32) + b_ref[...]
    logits_ref[...] = logits

    m = jnp.max(logits, axis=-1, keepdims=True)
    e = jnp.exp(logits - m)
    probs = e / jnp.sum(e, axis=-1, keepdims=True)
    probs_ref[...] = probs

    idx_ref[0, 0, :] = jnp.argmax(probs, axis=-1).astype(jnp.int32)


@functools.partial(jax.jit, static_argnums=())
def kernel(hidden_states, W, b):
    B, S, D = hidden_states.shape
    n_tok = B * S
    n_tiles = n_tok // _TS
    hs2 = hidden_states.reshape(n_tok, D)
    b2 = b.reshape(1, _E)

    logits, probs, idx = pl.pallas_call(
        _router_kernel,
        grid=(n_tiles,),
        in_specs=[
            pl.BlockSpec((_TS, _D), lambda t: (t, 0)),
            pl.BlockSpec((_D, _E), lambda t: (0, 0)),
            pl.BlockSpec((1, _E), lambda t: (0, 0)),
        ],
        out_specs=[
            pl.BlockSpec((_TS, _E), lambda t: (t, 0)),
            pl.BlockSpec((_TS, _E), lambda t: (t, 0)),
            pl.BlockSpec((1, 1, _TS), lambda t: (t, 0, 0)),
        ],
        out_shape=[
            jax.ShapeDtypeStruct((n_tok, _E), jnp.float32),
            jax.ShapeDtypeStruct((n_tok, _E), jnp.float32),
            jax.ShapeDtypeStruct((n_tiles, 1, _TS), jnp.int32),
        ],
    )(hs2, W, b2)

    return (idx.reshape(B, S), probs.reshape(B, S, _E), logits.reshape(B, S, _E))


# speedup vs baseline: 1.0237x; 1.0237x over previous
import functools

import jax
import jax.numpy as jnp
from jax.experimental import pallas as pl


_TS = 512
_D = 2048
_E = 64

_KS0 = 0
_KS1 = 42
_KS2 = _KS0 ^ _KS1 ^ 0x1BD11BDA
_ROTS = ((13, 15, 26, 6), (17, 29, 16, 24))
_ORDER = ((1, 2), (2, 0), (0, 1), (1, 2), (2, 0))
_KS = (_KS0, _KS1, _KS2)


def _rotl(x, r):
    return (x << jnp.uint32(r)) | (x >> jnp.uint32(32 - r))


_INJ0 = tuple(_KS[a] for a, _ in _ORDER)
_INJ1 = tuple((_KS[b] + i + 1) & 0xFFFFFFFF for i, (_, b) in enumerate(_ORDER))


def _router_kernel(hs_ref, w_ref, b_ref, logits_ref, probs_ref, idx_ref):
    t = pl.program_id(0)
    row = jax.lax.broadcasted_iota(jnp.uint32, (_TS, _D), 0)
    col = jax.lax.broadcasted_iota(jnp.uint32, (_TS, _D), 1)

    base = (t * (_TS * _D) + _KS1).astype(jnp.uint32)
    x1 = (row * jnp.uint32(_D) + col) + base
    x0 = x1
    x1 = _rotl(x1, _ROTS[0][0]) ^ x0
    first = True
    for i in range(5):
        for r in _ROTS[i % 2]:
            if first:
                first = False
                continue
            x0 = x0 + x1
            x1 = _rotl(x1, r)
            x1 = x1 ^ x0
        if _INJ0[i]:
            x0 = x0 + jnp.uint32(_INJ0[i])
        x1 = x1 + jnp.uint32(_INJ1[i])
    bits = x0 ^ x1

    u = jax.lax.bitcast_convert_type(
        (bits >> jnp.uint32(9)) | jnp.uint32(0x3F800000), jnp.float32) - 1.0
    noise = u * jnp.float32(-0.02) + jnp.float32(1.01)

    new_attr = hs_ref[...] * noise
    logits = jax.lax.dot_general(
        new_attr, w_ref[...], (((1,), (0,)), ((), ())),
        preferred_element_type=jnp.float32) + b_ref[...]
    logits_ref[...] = logits

    m = jnp.max(logits, axis=-1, keepdims=True)
    e = jnp.exp(logits - m)
    probs = e / jnp.sum(e, axis=-1, keepdims=True)
    probs_ref[...] = probs

    idx_ref[0, 0, :] = jnp.argmax(probs, axis=-1).astype(jnp.int32)


@functools.partial(jax.jit, static_argnums=())
def kernel(hidden_states, W, b):
    B, S, D = hidden_states.shape
    n_tok = B * S
    n_tiles = n_tok // _TS
    hs2 = hidden_states.reshape(n_tok, D)
    b2 = b.reshape(1, _E)

    logits, probs, idx = pl.pallas_call(
        _router_kernel,
        grid=(n_tiles,),
        in_specs=[
            pl.BlockSpec((_TS, _D), lambda t: (t, 0)),
            pl.BlockSpec((_D, _E), lambda t: (0, 0)),
            pl.BlockSpec((1, _E), lambda t: (0, 0)),
        ],
        out_specs=[
            pl.BlockSpec((_TS, _E), lambda t: (t, 0)),
            pl.BlockSpec((_TS, _E), lambda t: (t, 0)),
            pl.BlockSpec((1, 1, _TS), lambda t: (t, 0, 0)),
        ],
        out_shape=[
            jax.ShapeDtypeStruct((n_tok, _E), jnp.float32),
            jax.ShapeDtypeStruct((n_tok, _E), jnp.float32),
            jax.ShapeDtypeStruct((n_tiles, 1, _TS), jnp.int32),
        ],
    )(hs2, W, b2)

    return (idx.reshape(B, S), probs.reshape(B, S, _E), logits.reshape(B, S, _E))
